# baseline (speedup 1.0000x reference)
import functools

import jax
import jax.numpy as jnp
from jax import lax
from jax.experimental import pallas as pl
from jax.experimental.pallas import tpu as pltpu
from jax.experimental.pallas import tpu_sc as plsc

_NUM_OBJS = 151
_NUM_RELS = 51
_BATCH = 16384
_DPAD = 64
_NSUB = _DPAD // 16

_NC = 2
_NS = 16
_NW = _NC * _NS
_BPW = _BATCH // _NW
_SPW = _BPW * _NSUB
_GCH = 128
_NG = _SPW // _GCH
_PKW = _BPW * _NUM_RELS

_L = 16


_DNUMS = lax.GatherDimensionNumbers(
    offset_dims=(), collapsed_slice_dims=(0,), start_index_map=(0,))


def _dg(v, idx):
    return lax.gather(v, idx[:, None], dimension_numbers=_DNUMS,
                      slice_sizes=(1,),
                      mode=lax.GatherScatterMode.PROMISE_IN_BOUNDS)


def _freq_bias_body(lab_hbm, table_hbm, out_hbm,
                    lab_v, idx_v, sub_v, packed_v, sem):
    wid = lax.axis_index("s") * _NC + lax.axis_index("c")
    base = wid * _BPW

    pltpu.sync_copy(lab_hbm.at[pl.ds(base * 2, _BPW * 2)],
                    lab_v.at[pl.ds(0, _BPW * 2)])

    iota = lax.iota(jnp.int32, _L)
    part = iota % _NSUB
    e2 = (iota // _NSUB) * 2

    for q in range(_SPW // _L):
        v = lab_v[pl.ds(8 * q, _L)]
        l0 = _dg(v, e2)
        l1 = _dg(v, e2 + 1)
        idx_v[q // (_GCH // _L),
              pl.ds((q % (_GCH // _L)) * _L, _L)] = (
                  (l0 * _NUM_OBJS + l1) * _NSUB + part)

    copies = [
        pltpu.async_copy(table_hbm.at[idx_v.at[g]],
                         sub_v.at[pl.ds(g * _GCH, _GCH)], sem)
        for g in range(_NG)
    ]
    for cp in copies:
        cp.wait()

    sh3 = (iota + 3) % _L
    tail_lo = iota < (_L - 3)

    def depad_row(r, _):
        s2 = sub_v[r * _NSUB + 2]
        s3 = sub_v[r * _NSUB + 3]
        packed_v[pl.ds(r * _NUM_RELS, _L)] = sub_v[r * _NSUB]
        packed_v[pl.ds(r * _NUM_RELS + 16, _L)] = sub_v[r * _NSUB + 1]
        packed_v[pl.ds(r * _NUM_RELS + 32, _L)] = s2
        packed_v[pl.ds(r * _NUM_RELS + 35, _L)] = jnp.where(
            tail_lo, _dg(s2, sh3), _dg(s3, sh3))
        return _

    lax.fori_loop(0, _BPW, depad_row, 0)

    pltpu.sync_copy(packed_v.at[pl.ds(0, _PKW)],
                    out_hbm.at[pl.ds(wid * _PKW, _PKW)])


_freq_bias = functools.partial(
    pl.kernel,
    out_type=jax.ShapeDtypeStruct((_BATCH * _NUM_RELS,), jnp.float32),
    mesh=plsc.VectorSubcoreMesh(core_axis_name="c", subcore_axis_name="s"),
    compiler_params=pltpu.CompilerParams(use_tc_tiling_on_sc=False,
                                         needs_layout_passes=False),
    scratch_types=[
        pltpu.VMEM((_BPW * 2 + _L,), jnp.int32),
        pltpu.VMEM((_NG, _GCH), jnp.int32),
        pltpu.VMEM((_SPW, _L), jnp.float32),
        pltpu.VMEM((_PKW + _L,), jnp.float32),
        pltpu.SemaphoreType.DMA,
    ],
)(_freq_bias_body)


def kernel(labels, obj_baseline_weight):
    lab_flat = labels.astype(jnp.int32).reshape(-1)
    table_sub = jnp.pad(obj_baseline_weight,
                        ((0, 0), (0, _DPAD - _NUM_RELS))
                        ).reshape(_NUM_OBJS * _NUM_OBJS * _NSUB, 16)
    out_flat = _freq_bias(lab_flat, table_sub)
    return out_flat.reshape(_BATCH, _NUM_RELS)


# speedup vs baseline: 1.7208x; 1.7208x over previous
import functools

import jax
import jax.numpy as jnp
from jax import lax
from jax.experimental import pallas as pl
from jax.experimental.pallas import tpu as pltpu
from jax.experimental.pallas import tpu_sc as plsc

_NUM_OBJS = 151
_NUM_RELS = 51
_BATCH = 16384

_NC = 2
_NS = 16
_NW = _NC * _NS
_BPW = _BATCH // _NW


def _freq_bias_body(lab_hbm, table_hbm, out_hbm,
                    lab_sh, lab_s, rows_v, sem):
    wid = lax.axis_index("s") * _NC + lax.axis_index("c")
    base = wid * _BPW

    sid = lax.axis_index("s")
    pltpu.sync_copy(lab_hbm.at[pl.ds(base * 2, _BPW * 2)], lab_sh.at[sid])
    pltpu.sync_copy(lab_sh.at[sid], lab_s)

    def fire_row(i, _):
        r = lab_s[2 * i] * _NUM_OBJS + lab_s[2 * i + 1]
        pltpu.async_copy(table_hbm.at[pl.ds(r, 1)],
                         rows_v.at[pl.ds(i, 1)], sem)
        return _

    lax.fori_loop(0, _BPW, fire_row, 0)

    pltpu.make_async_copy(table_hbm.at[pl.ds(0, _BPW)], rows_v, sem).wait()

    pltpu.sync_copy(rows_v, out_hbm.at[pl.ds(base, _BPW)])


_freq_bias = functools.partial(
    pl.kernel,
    out_type=jax.ShapeDtypeStruct((_BATCH, _NUM_RELS), jnp.float32),
    mesh=plsc.VectorSubcoreMesh(core_axis_name="c", subcore_axis_name="s"),
    compiler_params=pltpu.CompilerParams(use_tc_tiling_on_sc=True),
    scratch_types=[
        pltpu.VMEM_SHARED((_NS, _BPW * 2), jnp.int32),
        pltpu.SMEM((_BPW * 2,), jnp.int32),
        pltpu.VMEM((_BPW, _NUM_RELS), jnp.float32),
        pltpu.SemaphoreType.DMA,
    ],
)(_freq_bias_body)


def kernel(labels, obj_baseline_weight):
    lab_flat = labels.astype(jnp.int32).reshape(-1)
    return _freq_bias(lab_flat, obj_baseline_weight)


# speedup vs baseline: 2.3102x; 1.3425x over previous
import functools

import jax
import jax.numpy as jnp
from jax import lax
from jax.experimental import pallas as pl
from jax.experimental.pallas import tpu as pltpu
from jax.experimental.pallas import tpu_sc as plsc

_NUM_OBJS = 151
_NUM_RELS = 51
_BATCH = 16384
_NROWS = _NUM_OBJS * _NUM_OBJS

_NC = 2
_NS = 16
_NW = _NC * _NS
_L = 16


def _gather_column(row_v, idx_v, col_v):
    def chunk(i, _):
        v = plsc.load_gather(row_v, [idx_v[pl.ds(i * _L, _L)]])
        col_v[pl.ds(i * _L, _L)] = v
        return _

    lax.fori_loop(0, _BATCH // _L, chunk, 0)


def _freq_bias_body(idx_hbm, tableT_hbm, outT_hbm,
                    idx_v, row0_v, row1_v, col0_v, col1_v, sem):
    wid = lax.axis_index("s") * _NC + lax.axis_index("c")

    cp_idx = pltpu.async_copy(idx_hbm, idx_v, sem)
    cp_r0 = pltpu.async_copy(tableT_hbm.at[wid], row0_v, sem)
    second = wid + _NW < _NUM_RELS

    @pl.when(second)
    def _():
        pltpu.async_copy(tableT_hbm.at[wid + _NW], row1_v, sem).wait()

    cp_idx.wait()
    cp_r0.wait()

    _gather_column(row0_v, idx_v, col0_v)
    pltpu.sync_copy(col0_v, outT_hbm.at[wid])

    @pl.when(second)
    def _():
        _gather_column(row1_v, idx_v, col1_v)
        pltpu.sync_copy(col1_v, outT_hbm.at[wid + _NW])


_freq_bias = functools.partial(
    pl.kernel,
    out_type=jax.ShapeDtypeStruct((_NUM_RELS, _BATCH), jnp.float32),
    mesh=plsc.VectorSubcoreMesh(core_axis_name="c", subcore_axis_name="s"),
    compiler_params=pltpu.CompilerParams(use_tc_tiling_on_sc=True,
                                         needs_layout_passes=False),
    scratch_types=[
        pltpu.VMEM((_BATCH,), jnp.int32),
        pltpu.VMEM((_NROWS,), jnp.float32),
        pltpu.VMEM((_NROWS,), jnp.float32),
        pltpu.VMEM((_BATCH,), jnp.float32),
        pltpu.VMEM((_BATCH,), jnp.float32),
        pltpu.SemaphoreType.DMA,
    ],
)(_freq_bias_body)


def kernel(labels, obj_baseline_weight):
    labels = labels.astype(jnp.int32)
    pair_idx = labels[:, 0] * _NUM_OBJS + labels[:, 1]
    outT = _freq_bias(pair_idx, obj_baseline_weight.T)
    return outT.T
